# baseline (speedup 1.0000x reference)
import jax
import jax.numpy as jnp
from jax.experimental import pallas as pl
from jax.experimental.pallas import tpu as pltpu

_EMBED = 32
_H1 = 64
_BLK = 2048


def _mlp_body(x_ref, w0a_ref, w0b_ref, b0_ref, w1t_ref, b1_ref, out_ref):
    gus = x_ref[0]
    gis = x_ref[1]
    h = (
        jnp.dot(gus, w0a_ref[...], preferred_element_type=jnp.float32)
        + jnp.dot(gis, w0b_ref[...], preferred_element_type=jnp.float32)
        + b0_ref[...]
    )
    h = jnp.maximum(h, 0.0)
    out_ref[...] = (
        jnp.sum(h * w1t_ref[...], axis=1, keepdims=True) + b1_ref[...]
    )


def kernel(inputs, W0, b0, W1, b1):
    batch = inputs.shape[1]
    grid = batch // _BLK
    w0a = W0[:_EMBED]
    w0b = W0[_EMBED:]
    b0r = b0.reshape(1, _H1)
    w1t = W1.reshape(1, _H1)
    b1r = b1.reshape(1, 1)

    return pl.pallas_call(
        _mlp_body,
        grid=(grid,),
        in_specs=[
            pl.BlockSpec((2, _BLK, _EMBED), lambda i: (0, i, 0)),
            pl.BlockSpec((_EMBED, _H1), lambda i: (0, 0)),
            pl.BlockSpec((_EMBED, _H1), lambda i: (0, 0)),
            pl.BlockSpec((1, _H1), lambda i: (0, 0)),
            pl.BlockSpec((1, _H1), lambda i: (0, 0)),
            pl.BlockSpec((1, 1), lambda i: (0, 0)),
        ],
        out_specs=pl.BlockSpec((_BLK, 1), lambda i: (i, 0)),
        out_shape=jax.ShapeDtypeStruct((batch, 1), jnp.float32),
        compiler_params=pltpu.CompilerParams(
            dimension_semantics=("arbitrary",),
        ),
    )(inputs, w0a, w0b, b0r, w1t, b1r)
